# baseline (device time: 34107 ns/iter reference)
import jax
import jax.numpy as jnp
from jax import lax
from jax.experimental import pallas as pl
from jax.experimental.pallas import tpu as pltpu

T_LOC = 256
D = 512
E_LOC = 2
F = 1024


def kernel(x, router, W1, W2):
    def body(x_ref, router_ref, w1_ref, w2_ref, out_ref,
             x_other, router_other, part_send, part_recv,
             send_sems, recv_sems):
        my_x = lax.axis_index("x")
        my_y = lax.axis_index("y")
        peer = (my_x, 1 - my_y)

        rdma_x = pltpu.make_async_remote_copy(
            src_ref=x_ref, dst_ref=x_other,
            send_sem=send_sems.at[0], recv_sem=recv_sems.at[0],
            device_id=peer, device_id_type=pl.DeviceIdType.MESH,
        )
        rdma_x.start()
        rdma_r = pltpu.make_async_remote_copy(
            src_ref=router_ref, dst_ref=router_other,
            send_sem=send_sems.at[1], recv_sem=recv_sems.at[1],
            device_id=peer, device_id_type=pl.DeviceIdType.MESH,
        )
        rdma_r.start()
        rdma_r.wait()
        rdma_x.wait()

        is_lo = my_y == 0
        x_mine = x_ref[:, :]
        x_peer = x_other[:, :]
        full_x = jnp.where(
            is_lo,
            jnp.concatenate([x_mine, x_peer], axis=0),
            jnp.concatenate([x_peer, x_mine], axis=0),
        )

        g_mine = jnp.dot(full_x, router_ref[:, :],
                         preferred_element_type=jnp.float32,
                         precision=lax.Precision.HIGHEST)
        g_peer = jnp.dot(full_x, router_other[:, :],
                         preferred_element_type=jnp.float32,
                         precision=lax.Precision.HIGHEST)
        gates = jnp.where(
            is_lo,
            jnp.concatenate([g_mine, g_peer], axis=1),
            jnp.concatenate([g_peer, g_mine], axis=1),
        )

        m1 = jnp.max(gates, axis=1, keepdims=True)
        mask1 = gates >= m1
        rest = jnp.where(mask1, -jnp.inf, gates)
        m2 = jnp.max(rest, axis=1, keepdims=True)
        mask2 = rest >= m2
        z = jnp.exp(m2 - m1)
        w1v = 1.0 / (1.0 + z)
        w2v = z / (1.0 + z)
        wgt = w1v * mask1.astype(jnp.float32) + w2v * mask2.astype(jnp.float32)

        e_iota = lax.broadcasted_iota(jnp.int32, (2 * T_LOC, 2 * E_LOC), 1)
        acc = jnp.zeros((2 * T_LOC, D), jnp.float32)
        for l in range(E_LOC):
            e_global = E_LOC * my_y + l
            w_e = jnp.sum(
                jnp.where(e_iota == e_global, wgt, 0.0), axis=1, keepdims=True
            )
            h = jnp.maximum(
                jnp.dot(full_x.astype(jnp.bfloat16),
                        w1_ref[l].astype(jnp.bfloat16),
                        preferred_element_type=jnp.float32), 0.0)
            y_e = jnp.dot(h.astype(jnp.bfloat16),
                          w2_ref[l].astype(jnp.bfloat16),
                          preferred_element_type=jnp.float32)
            acc = acc + y_e * w_e

        mine = jnp.where(is_lo, acc[:T_LOC], acc[T_LOC:])
        to_peer = jnp.where(is_lo, acc[T_LOC:], acc[:T_LOC])
        part_send[:, :] = to_peer
        rdma_p = pltpu.make_async_remote_copy(
            src_ref=part_send, dst_ref=part_recv,
            send_sem=send_sems.at[2], recv_sem=recv_sems.at[2],
            device_id=peer, device_id_type=pl.DeviceIdType.MESH,
        )
        rdma_p.start()
        rdma_p.wait()
        out_ref[:, :] = mine + part_recv[:, :]

    return pl.pallas_call(
        body,
        out_shape=jax.ShapeDtypeStruct((T_LOC, D), jnp.float32),
        in_specs=[pl.BlockSpec(memory_space=pltpu.VMEM)] * 4,
        out_specs=pl.BlockSpec(memory_space=pltpu.VMEM),
        scratch_shapes=[
            pltpu.VMEM((T_LOC, D), jnp.float32),
            pltpu.VMEM((D, E_LOC), jnp.float32),
            pltpu.VMEM((T_LOC, D), jnp.float32),
            pltpu.VMEM((T_LOC, D), jnp.float32),
            pltpu.SemaphoreType.DMA((3,)),
            pltpu.SemaphoreType.DMA((3,)),
        ],
    )(x, router, W1, W2)


# device time: 33305 ns/iter; 1.0241x vs baseline; 1.0241x over previous
import jax
import jax.numpy as jnp
from jax import lax
from jax.experimental import pallas as pl
from jax.experimental.pallas import tpu as pltpu

T_LOC = 256
D = 512
E_LOC = 2
F = 1024


def _top2_weights(gates):
    m1 = jnp.max(gates, axis=1, keepdims=True)
    mask1 = gates >= m1
    rest = jnp.where(mask1, -jnp.inf, gates)
    m2 = jnp.max(rest, axis=1, keepdims=True)
    mask2 = rest >= m2
    z = jnp.exp(m2 - m1)
    w1v = 1.0 / (1.0 + z)
    w2v = z / (1.0 + z)
    return w1v * mask1.astype(jnp.float32) + w2v * mask2.astype(jnp.float32)


def kernel(x, router, W1, W2):
    def body(x_ref, router_ref, w1_ref, w2_ref, out_ref,
             x_other, router_other, part_send, part_recv,
             send_sems, recv_sems):
        my_x = lax.axis_index("x")
        my_y = lax.axis_index("y")
        peer = (my_x, 1 - my_y)

        rdma_x = pltpu.make_async_remote_copy(
            src_ref=x_ref, dst_ref=x_other,
            send_sem=send_sems.at[0], recv_sem=recv_sems.at[0],
            device_id=peer, device_id_type=pl.DeviceIdType.MESH,
        )
        rdma_x.start()
        rdma_r = pltpu.make_async_remote_copy(
            src_ref=router_ref, dst_ref=router_other,
            send_sem=send_sems.at[1], recv_sem=recv_sems.at[1],
            device_id=peer, device_id_type=pl.DeviceIdType.MESH,
        )
        rdma_r.start()

        rdma_r.wait()
        x_mine = x_ref[:, :]
        gates_mine = jnp.concatenate([
            jnp.dot(x_mine, router_ref[:, :],
                    preferred_element_type=jnp.float32,
                    precision=lax.Precision.HIGHEST),
            jnp.dot(x_mine, router_other[:, :],
                    preferred_element_type=jnp.float32,
                    precision=lax.Precision.HIGHEST),
        ], axis=1)
        wgt_mine = _top2_weights(gates_mine)

        rdma_x.wait()
        x_peer = x_other[:, :]
        gates_peer = jnp.concatenate([
            jnp.dot(x_peer, router_ref[:, :],
                    preferred_element_type=jnp.float32,
                    precision=lax.Precision.HIGHEST),
            jnp.dot(x_peer, router_other[:, :],
                    preferred_element_type=jnp.float32,
                    precision=lax.Precision.HIGHEST),
        ], axis=1)
        wgt_peer = _top2_weights(gates_peer)

        acc_peer = jnp.zeros((T_LOC, D), jnp.float32)
        for l in range(E_LOC):
            h = jnp.maximum(
                jnp.dot(x_peer, w1_ref[l],
                        preferred_element_type=jnp.float32), 0.0)
            y_e = jnp.dot(h, w2_ref[l], preferred_element_type=jnp.float32)
            acc_peer = acc_peer + y_e * wgt_peer[:, l:l + 1]

        part_send[:, :] = acc_peer
        rdma_p = pltpu.make_async_remote_copy(
            src_ref=part_send, dst_ref=part_recv,
            send_sem=send_sems.at[2], recv_sem=recv_sems.at[2],
            device_id=peer, device_id_type=pl.DeviceIdType.MESH,
        )
        rdma_p.start()

        acc_mine = jnp.zeros((T_LOC, D), jnp.float32)
        for l in range(E_LOC):
            h = jnp.maximum(
                jnp.dot(x_mine, w1_ref[l],
                        preferred_element_type=jnp.float32), 0.0)
            y_e = jnp.dot(h, w2_ref[l], preferred_element_type=jnp.float32)
            acc_mine = acc_mine + y_e * wgt_mine[:, l:l + 1]

        rdma_p.wait()
        out_ref[:, :] = acc_mine + part_recv[:, :]

    return pl.pallas_call(
        body,
        out_shape=jax.ShapeDtypeStruct((T_LOC, D), jnp.float32),
        in_specs=[pl.BlockSpec(memory_space=pltpu.VMEM)] * 4,
        out_specs=pl.BlockSpec(memory_space=pltpu.VMEM),
        scratch_shapes=[
            pltpu.VMEM((T_LOC, D), jnp.float32),
            pltpu.VMEM((D, E_LOC), jnp.float32),
            pltpu.VMEM((T_LOC, D), jnp.float32),
            pltpu.VMEM((T_LOC, D), jnp.float32),
            pltpu.SemaphoreType.DMA((3,)),
            pltpu.SemaphoreType.DMA((3,)),
        ],
    )(x, router, W1, W2)
